# baseline (speedup 1.0000x reference)
import functools

import jax
import jax.numpy as jnp
from jax import lax
from jax.experimental import pallas as pl
from jax.experimental.pallas import tpu as pltpu
from jax.experimental.pallas import tpu_sc as plsc

VOCAB = 1000000
DIM = 64
SEQ = 200
BATCH = 4096

NUM_CORES = 2
NUM_SUBCORES = 16
NW = NUM_CORES * NUM_SUBCORES
TOTAL = BATCH * SEQ
PER_W = TOTAL // NW
SEQ_PER_W = BATCH // NW
CHUNK = SEQ
NCHUNK = PER_W // CHUNK
SPLIT_A = 128
SPLIT_B = CHUNK - SPLIT_A

_mesh = plsc.VectorSubcoreMesh(
    core_axis_name="c", subcore_axis_name="s",
    num_cores=NUM_CORES, num_subcores=NUM_SUBCORES)


@functools.partial(
    pl.kernel,
    out_type=[
        jax.ShapeDtypeStruct((TOTAL, DIM), jnp.float32),
        jax.ShapeDtypeStruct((BATCH,), jnp.int32),
    ],
    mesh=_mesh,
    compiler_params=pltpu.CompilerParams(
        needs_layout_passes=False, use_tc_tiling_on_sc=False),
    scratch_types=[
        pltpu.VMEM((PER_W,), jnp.int32),
        pltpu.VMEM((SEQ, DIM), jnp.float32),
        pltpu.VMEM((CHUNK, DIM), jnp.float32),
        pltpu.VMEM((CHUNK, DIM), jnp.float32),
        pltpu.VMEM((CHUNK, DIM), jnp.float32),
        pltpu.VMEM((CHUNK, DIM), jnp.float32),
        pltpu.VMEM((SEQ_PER_W,), jnp.int32),
        pltpu.SemaphoreType.DMA,
        pltpu.SemaphoreType.DMA,
        pltpu.SemaphoreType.DMA,
        pltpu.SemaphoreType.DMA,
    ],
)
def _emb_kernel(table_hbm, idx_hbm, pe_hbm, out_hbm, pad_hbm,
                idx_v, pe_v, in0, in1, out0, out1, pad_v,
                gsem0, gsem1, ssem0, ssem1):
    cid = lax.axis_index("c")
    sid = lax.axis_index("s")
    wid = sid * NUM_CORES + cid
    row_base = wid * PER_W

    pltpu.sync_copy(idx_hbm.at[wid], idx_v)

    def gather_start(g, buf, sem):
        pltpu.make_async_copy(
            table_hbm.at[idx_v.at[pl.ds(g * CHUNK, SPLIT_A)]],
            buf.at[pl.ds(0, SPLIT_A)], sem).start()
        pltpu.make_async_copy(
            table_hbm.at[idx_v.at[pl.ds(g * CHUNK + SPLIT_A, SPLIT_B)]],
            buf.at[pl.ds(SPLIT_A, SPLIT_B)], sem).start()

    def gather_wait(g, buf, sem):
        pltpu.make_async_copy(
            table_hbm.at[idx_v.at[pl.ds(g * CHUNK, SPLIT_A)]],
            buf.at[pl.ds(0, SPLIT_A)], sem).wait()
        pltpu.make_async_copy(
            table_hbm.at[idx_v.at[pl.ds(g * CHUNK + SPLIT_A, SPLIT_B)]],
            buf.at[pl.ds(SPLIT_A, SPLIT_B)], sem).wait()

    def scatter_start(g, buf, sem):
        pltpu.make_async_copy(
            buf, out_hbm.at[pl.ds(row_base + g * CHUNK, CHUNK)], sem).start()

    def scatter_wait(g, buf, sem):
        pltpu.make_async_copy(
            buf, out_hbm.at[pl.ds(row_base + g * CHUNK, CHUNK)], sem).wait()

    gather_start(0, in0, gsem0)
    gather_start(1, in1, gsem1)

    pltpu.sync_copy(pe_hbm, pe_v)

    iota16 = lax.iota(jnp.int32, 16)

    def grp_body(grp, _):
        s0 = grp * 16

        def p_body(p, best):
            flat = (s0 * SEQ + p) + SEQ * iota16
            tok = plsc.load_gather(idx_v, [flat])
            return jnp.maximum(best, jnp.where(tok != 0, p, 0))

        best = lax.fori_loop(0, SEQ, p_body, jnp.zeros((16,), jnp.int32))
        pad_v[pl.ds(s0, 16)] = best + 1
        return 0

    lax.fori_loop(0, SEQ_PER_W // 16, grp_body, 0)
    pltpu.sync_copy(pad_v, pad_hbm.at[pl.ds(wid * SEQ_PER_W, SEQ_PER_W)])

    def compute(inb, outb):
        @plsc.parallel_loop(0, CHUNK, 1, unroll=4)
        def _(p):
            for c in range(DIM // 16):
                s = pl.ds(16 * c, 16)
                outb[p, s] = inb[p, s] * 8.0 + pe_v[p, s]

    def chunk_phase(g, inb, outb, gsem, ssem):
        gather_wait(g, inb, gsem)

        @pl.when(g >= 2)
        def _():
            scatter_wait(g, outb, ssem)

        compute(inb, outb)

        @pl.when(g + 2 < NCHUNK)
        def _():
            gather_start(g + 2, inb, gsem)

        scatter_start(g, outb, ssem)

    def loop_body(gg, _):
        chunk_phase(2 * gg, in0, out0, gsem0, ssem0)
        chunk_phase(2 * gg + 1, in1, out1, gsem1, ssem1)
        return 0

    lax.fori_loop(0, NCHUNK // 2, loop_body, 0)

    scatter_wait(NCHUNK - 2, out0, ssem0)
    scatter_wait(NCHUNK - 1, out1, ssem1)


def kernel(token_tensor, table, PE):
    idx = token_tensor.astype(jnp.int32).reshape(NW, PER_W)
    out_flat, pad_idxs = _emb_kernel(table, idx, PE)
    return out_flat.reshape(BATCH, SEQ, DIM), pad_idxs
